# Initial kernel scaffold; baseline (speedup 1.0000x reference)
import jax
import jax.numpy as jnp
from jax.experimental import pallas as pl


def kernel(x, prev_weight, enc_W, enc_b, dec_W, alpha):
    raise NotImplementedError("write your pallas kernel here")


# speedup vs baseline: 11.8332x; 11.8332x over previous
import jax
import jax.numpy as jnp
from jax.experimental import pallas as pl

TOPK = 50
ROW_BLOCK = 512


def _fused_kernel(x_ref, encw_ref, encb_ref, prevw_ref, decw_ref, alpha_ref,
                  mod_ref, latent_ref, w_ref):
    x = x_ref[...]
    encw = encw_ref[...]
    latent = jax.lax.dot_general(
        x, encw, (((1,), (1,)), ((), ())), preferred_element_type=jnp.float32)
    latent = latent + encb_ref[...]
    latent_ref[...] = latent

    abits = jax.lax.bitcast_convert_type(latent, jnp.int32) & jnp.int32(0x7FFFFFFF)

    def body(i, t):
        cand = t | jax.lax.shift_left(jnp.int32(1), 30 - i)
        cnt = jnp.sum((abits >= cand).astype(jnp.int32), axis=1, keepdims=True)
        return jnp.where(cnt >= TOPK, cand, t)

    t0 = jnp.zeros((x.shape[0], 1), jnp.int32)
    thr = jax.lax.fori_loop(0, 31, body, t0)

    gt = abits > thr
    eq = abits == thr
    cnt_gt = jnp.sum(gt.astype(jnp.int32), axis=1, keepdims=True)
    need = (TOPK - cnt_gt).astype(jnp.float32)
    dl = abits.shape[1]
    ii = jax.lax.broadcasted_iota(jnp.int32, (dl, dl), 0)
    jj = jax.lax.broadcasted_iota(jnp.int32, (dl, dl), 1)
    tri = (ii < jj).astype(jnp.float32)
    rank = jax.lax.dot_general(
        eq.astype(jnp.float32), tri, (((1,), (0,)), ((), ())),
        preferred_element_type=jnp.float32)
    mask = gt | (eq & (rank < need))

    gated = jnp.where(mask, latent, 0.0)

    alpha = alpha_ref[0, 0]
    w = prevw_ref[...] + alpha * (decw_ref[...] - prevw_ref[...])

    @pl.when(pl.program_id(0) == 0)
    def _():
        w_ref[...] = w

    mod_ref[...] = jax.lax.dot_general(
        gated, w, (((1,), (1,)), ((), ())), preferred_element_type=jnp.float32)


def kernel(x, prev_weight, enc_W, enc_b, dec_W, alpha):
    B, L, D = x.shape
    Dl = enc_W.shape[0]
    N = B * L
    R = ROW_BLOCK
    x_flat = x.reshape(N, D)
    mod_flat, latent, W = pl.pallas_call(
        _fused_kernel,
        grid=(N // R,),
        in_specs=[
            pl.BlockSpec((R, D), lambda i: (i, 0)),
            pl.BlockSpec((Dl, D), lambda i: (0, 0)),
            pl.BlockSpec((1, Dl), lambda i: (0, 0)),
            pl.BlockSpec((D, Dl), lambda i: (0, 0)),
            pl.BlockSpec((D, Dl), lambda i: (0, 0)),
            pl.BlockSpec((1, 1), lambda i: (0, 0)),
        ],
        out_specs=[
            pl.BlockSpec((R, D), lambda i: (i, 0)),
            pl.BlockSpec((R, Dl), lambda i: (i, 0)),
            pl.BlockSpec((D, Dl), lambda i: (0, 0)),
        ],
        out_shape=[
            jax.ShapeDtypeStruct((N, D), jnp.float32),
            jax.ShapeDtypeStruct((N, Dl), jnp.float32),
            jax.ShapeDtypeStruct((D, Dl), jnp.float32),
        ],
    )(x_flat, enc_W, enc_b.reshape(1, Dl), prev_weight, dec_W,
      jnp.asarray(alpha, jnp.float32).reshape(1, 1))
    return (mod_flat.reshape(B, L, D), latent, W)


# speedup vs baseline: 25.3883x; 2.1455x over previous
import jax
import jax.numpy as jnp
from jax.experimental import pallas as pl

TOPK = 50
ROW_BLOCK = 512


def _fused_kernel(x_ref, encw_ref, encb_ref, prevw_ref, decw_ref, alpha_ref,
                  tri_ref, mod_ref, latent_ref, w_ref):
    x = x_ref[...]
    encw = encw_ref[...]
    latent_t = jax.lax.dot_general(
        encw, x, (((1,), (1,)), ((), ())), preferred_element_type=jnp.float32)
    latent_t = latent_t + encb_ref[...]
    latent_ref[...] = latent_t.T

    abits = jax.lax.bitcast_convert_type(latent_t, jnp.int32) & jnp.int32(0x7FFFFFFF)

    t = jnp.zeros((1, abits.shape[1]), jnp.int32)
    for b in range(30, -1, -1):
        cand = t | jnp.int32(1 << b)
        cnt = jnp.sum((abits >= cand).astype(jnp.int32), axis=0, keepdims=True)
        t = jnp.where(cnt >= TOPK, cand, t)

    gt = abits > t
    eq = abits == t
    cnt_gt = jnp.sum(gt.astype(jnp.int32), axis=0, keepdims=True)
    need = (TOPK - cnt_gt).astype(jnp.float32)
    rank = jax.lax.dot_general(
        tri_ref[...], eq.astype(jnp.float32), (((1,), (0,)), ((), ())),
        preferred_element_type=jnp.float32)
    mask = gt | (eq & (rank < need))

    gated_t = jnp.where(mask, latent_t, 0.0)

    alpha = alpha_ref[0, 0]
    w = prevw_ref[...] + alpha * (decw_ref[...] - prevw_ref[...])

    @pl.when(pl.program_id(0) == 0)
    def _():
        w_ref[...] = w

    mod_ref[...] = jax.lax.dot_general(
        gated_t, w, (((0,), (1,)), ((), ())), preferred_element_type=jnp.float32)


def kernel(x, prev_weight, enc_W, enc_b, dec_W, alpha):
    B, L, D = x.shape
    Dl = enc_W.shape[0]
    N = B * L
    R = ROW_BLOCK
    x_flat = x.reshape(N, D)
    ll = jnp.arange(Dl, dtype=jnp.int32)
    tri = (ll[None, :] < ll[:, None]).astype(jnp.float32)
    mod_flat, latent, W = pl.pallas_call(
        _fused_kernel,
        grid=(N // R,),
        in_specs=[
            pl.BlockSpec((R, D), lambda i: (i, 0)),
            pl.BlockSpec((Dl, D), lambda i: (0, 0)),
            pl.BlockSpec((Dl, 1), lambda i: (0, 0)),
            pl.BlockSpec((D, Dl), lambda i: (0, 0)),
            pl.BlockSpec((D, Dl), lambda i: (0, 0)),
            pl.BlockSpec((1, 1), lambda i: (0, 0)),
            pl.BlockSpec((Dl, Dl), lambda i: (0, 0)),
        ],
        out_specs=[
            pl.BlockSpec((R, D), lambda i: (i, 0)),
            pl.BlockSpec((R, Dl), lambda i: (i, 0)),
            pl.BlockSpec((D, Dl), lambda i: (0, 0)),
        ],
        out_shape=[
            jax.ShapeDtypeStruct((N, D), jnp.float32),
            jax.ShapeDtypeStruct((N, Dl), jnp.float32),
            jax.ShapeDtypeStruct((D, Dl), jnp.float32),
        ],
    )(x_flat, enc_W, enc_b.reshape(Dl, 1), prev_weight, dec_W,
      jnp.asarray(alpha, jnp.float32).reshape(1, 1), tri)
    return (mod_flat.reshape(B, L, D), latent, W)


# speedup vs baseline: 28.1461x; 1.1086x over previous
import jax
import jax.numpy as jnp
from jax.experimental import pallas as pl

TOPK = 50
ROW_BLOCK = 512


def _sum_sublanes(v):
    while v.shape[0] > 8:
        h = v.shape[0] // 2
        v = v[:h] + v[h:]
    return jnp.sum(v, axis=0, keepdims=True)


def _fused_kernel(x_ref, encw_ref, encb_ref, prevw_ref, decw_ref, alpha_ref,
                  tri_ref, mod_ref, latent_ref, w_ref):
    x = x_ref[...]
    encw = encw_ref[...]
    latent_t = jax.lax.dot_general(
        encw, x, (((1,), (1,)), ((), ())), preferred_element_type=jnp.float32)
    latent_t = latent_t + encb_ref[...]
    latent_ref[...] = latent_t.T

    abits = jax.lax.bitcast_convert_type(latent_t, jnp.int32) & jnp.int32(0x7FFFFFFF)

    t = jnp.zeros((1, abits.shape[1]), jnp.int32)
    for b in range(30, -1, -1):
        cand = t | jnp.int32(1 << b)
        cnt = _sum_sublanes((abits >= cand).astype(jnp.int32))
        t = jnp.where(cnt >= TOPK, cand, t)

    gt = abits > t
    eq = abits == t
    cnt_gt = _sum_sublanes(gt.astype(jnp.int32))
    need = (TOPK - cnt_gt).astype(jnp.float32)
    rank = jax.lax.dot_general(
        tri_ref[...], eq.astype(jnp.float32), (((1,), (0,)), ((), ())),
        preferred_element_type=jnp.float32)
    mask = gt | (eq & (rank < need))

    gated_t = jnp.where(mask, latent_t, 0.0)

    alpha = alpha_ref[0, 0]
    w = prevw_ref[...] + alpha * (decw_ref[...] - prevw_ref[...])

    @pl.when(pl.program_id(0) == 0)
    def _():
        w_ref[...] = w

    mod_ref[...] = jax.lax.dot_general(
        gated_t, w, (((0,), (1,)), ((), ())), preferred_element_type=jnp.float32)


def kernel(x, prev_weight, enc_W, enc_b, dec_W, alpha):
    B, L, D = x.shape
    Dl = enc_W.shape[0]
    N = B * L
    R = ROW_BLOCK
    x_flat = x.reshape(N, D)
    ll = jnp.arange(Dl, dtype=jnp.int32)
    tri = (ll[None, :] < ll[:, None]).astype(jnp.float32)
    mod_flat, latent, W = pl.pallas_call(
        _fused_kernel,
        grid=(N // R,),
        in_specs=[
            pl.BlockSpec((R, D), lambda i: (i, 0)),
            pl.BlockSpec((Dl, D), lambda i: (0, 0)),
            pl.BlockSpec((Dl, 1), lambda i: (0, 0)),
            pl.BlockSpec((D, Dl), lambda i: (0, 0)),
            pl.BlockSpec((D, Dl), lambda i: (0, 0)),
            pl.BlockSpec((1, 1), lambda i: (0, 0)),
            pl.BlockSpec((Dl, Dl), lambda i: (0, 0)),
        ],
        out_specs=[
            pl.BlockSpec((R, D), lambda i: (i, 0)),
            pl.BlockSpec((R, Dl), lambda i: (i, 0)),
            pl.BlockSpec((D, Dl), lambda i: (0, 0)),
        ],
        out_shape=[
            jax.ShapeDtypeStruct((N, D), jnp.float32),
            jax.ShapeDtypeStruct((N, Dl), jnp.float32),
            jax.ShapeDtypeStruct((D, Dl), jnp.float32),
        ],
    )(x_flat, enc_W, enc_b.reshape(Dl, 1), prev_weight, dec_W,
      jnp.asarray(alpha, jnp.float32).reshape(1, 1), tri)
    return (mod_flat.reshape(B, L, D), latent, W)


# speedup vs baseline: 32.0026x; 1.1370x over previous
import jax
import jax.numpy as jnp
import numpy as np
from jax.experimental import pallas as pl

TOPK = 50
ROW_BLOCK = 512


def _sum_sublanes(v):
    while v.shape[0] > 16:
        h = v.shape[0] // 2
        v = v[:h] + v[h:]
    return jnp.sum(v, axis=0, keepdims=True)


def _fused_kernel(x_ref, encw_ref, encb_ref, prevw_ref, decw_ref, alpha_ref,
                  tri_ref, mod_ref, latent_ref, w_ref):
    x = x_ref[...]
    encw = encw_ref[...]
    latent_t = jax.lax.dot_general(
        encw, x, (((1,), (1,)), ((), ())), preferred_element_type=jnp.float32)
    latent_t = latent_t + encb_ref[...]
    latent_ref[...] = latent_t.T

    abits = jax.lax.bitcast_convert_type(latent_t, jnp.int32) & jnp.int32(0x7FFFFFFF)
    rcols = abits.shape[1]

    hi = (abits >> 16).astype(jnp.int16)
    t1 = jnp.zeros((1, rcols), jnp.int16)
    for b in range(14, -1, -1):
        cand = t1 | jnp.int16(1 << b)
        cnt = _sum_sublanes((hi >= cand).astype(jnp.int16))
        m = (cnt - jnp.int16(TOPK)) >> 15
        t1 = cand ^ ((cand ^ t1) & m)
    cnt_hi_gt = _sum_sublanes((hi > t1).astype(jnp.int16))
    k2 = jnp.int16(TOPK) - cnt_hi_gt

    lo = (abits ^ jnp.int32(0x8000)).astype(jnp.int16)
    loa = jnp.where(hi == t1, lo, jnp.int16(-32768))
    t2 = jnp.full((1, rcols), -32768, dtype=jnp.int16)
    for b in range(15, -1, -1):
        cand = t2 + jnp.int16(np.int16(np.uint16(1 << b)))
        cnt = _sum_sublanes((loa >= cand).astype(jnp.int16))
        m = (cnt - k2) >> 15
        t2 = cand ^ ((cand ^ t2) & m)

    thr = (t1.astype(jnp.int32) << 16) | ((t2.astype(jnp.int32) ^ 0x8000) & 0xFFFF)

    gt = abits > thr
    eq = abits == thr
    cnt_gt = _sum_sublanes(gt.astype(jnp.int32))
    need = (TOPK - cnt_gt).astype(jnp.float32)
    rank = jax.lax.dot_general(
        tri_ref[...], eq.astype(jnp.float32), (((1,), (0,)), ((), ())),
        preferred_element_type=jnp.float32)
    mask = gt | (eq & (rank < need))

    gated_t = jnp.where(mask, latent_t, 0.0)

    @pl.when(pl.program_id(0) == 0)
    def _():
        alpha = alpha_ref[0, 0]
        w_ref[...] = prevw_ref[...] + alpha * (decw_ref[...] - prevw_ref[...])
    w = w_ref[...]

    mod_ref[...] = jax.lax.dot_general(
        gated_t, w, (((0,), (1,)), ((), ())), preferred_element_type=jnp.float32)


def kernel(x, prev_weight, enc_W, enc_b, dec_W, alpha):
    B, L, D = x.shape
    Dl = enc_W.shape[0]
    N = B * L
    R = ROW_BLOCK
    x_flat = x.reshape(N, D)
    ll = jnp.arange(Dl, dtype=jnp.int32)
    tri = (ll[None, :] < ll[:, None]).astype(jnp.float32)
    mod_flat, latent, W = pl.pallas_call(
        _fused_kernel,
        grid=(N // R,),
        in_specs=[
            pl.BlockSpec((R, D), lambda i: (i, 0)),
            pl.BlockSpec((Dl, D), lambda i: (0, 0)),
            pl.BlockSpec((Dl, 1), lambda i: (0, 0)),
            pl.BlockSpec((D, Dl), lambda i: (0, 0)),
            pl.BlockSpec((D, Dl), lambda i: (0, 0)),
            pl.BlockSpec((1, 1), lambda i: (0, 0)),
            pl.BlockSpec((Dl, Dl), lambda i: (0, 0)),
        ],
        out_specs=[
            pl.BlockSpec((R, D), lambda i: (i, 0)),
            pl.BlockSpec((R, Dl), lambda i: (i, 0)),
            pl.BlockSpec((D, Dl), lambda i: (0, 0)),
        ],
        out_shape=[
            jax.ShapeDtypeStruct((N, D), jnp.float32),
            jax.ShapeDtypeStruct((N, Dl), jnp.float32),
            jax.ShapeDtypeStruct((D, Dl), jnp.float32),
        ],
    )(x_flat, enc_W, enc_b.reshape(Dl, 1), prev_weight, dec_W,
      jnp.asarray(alpha, jnp.float32).reshape(1, 1), tri)
    return (mod_flat.reshape(B, L, D), latent, W)


# speedup vs baseline: 37.6756x; 1.1773x over previous
import jax
import jax.numpy as jnp
import numpy as np
from jax.experimental import pallas as pl

TOPK = 50
ROW_BLOCK = 1024


def _sum_sublanes(v):
    while v.shape[0] > 16:
        h = v.shape[0] // 2
        v = v[:h] + v[h:]
    return jnp.sum(v, axis=0, keepdims=True)


def _fused_kernel(x_ref, encw_ref, encb_ref, prevw_ref, decw_ref, alpha_ref,
                  tri_ref, mod_ref, latent_ref, w_ref):
    x = x_ref[...]
    encw = encw_ref[...]
    latent_t = jax.lax.dot_general(
        encw, x, (((1,), (1,)), ((), ())), preferred_element_type=jnp.float32)
    latent_t = latent_t + encb_ref[...]
    latent_ref[...] = latent_t.T

    abits = jax.lax.bitcast_convert_type(latent_t, jnp.int32) & jnp.int32(0x7FFFFFFF)
    rcols = abits.shape[1]

    hi = (abits >> 16).astype(jnp.int16)
    t1 = jnp.zeros((1, rcols), jnp.int16)
    for b in range(14, -1, -1):
        cand = t1 | jnp.int16(1 << b)
        cnt = _sum_sublanes((hi >= cand).astype(jnp.int16))
        m = (cnt - jnp.int16(TOPK)) >> 15
        t1 = cand ^ ((cand ^ t1) & m)
    cnt_hi_gt = _sum_sublanes((hi > t1).astype(jnp.int16))
    k2 = jnp.int16(TOPK) - cnt_hi_gt

    lo = (abits ^ jnp.int32(0x8000)).astype(jnp.int16)
    loa = jnp.where(hi == t1, lo, jnp.int16(-32768))
    t2 = jnp.full((1, rcols), -32768, dtype=jnp.int16)
    for b in range(15, -1, -1):
        cand = t2 + jnp.int16(np.int16(np.uint16(1 << b)))
        cnt = _sum_sublanes((loa >= cand).astype(jnp.int16))
        m = (cnt - k2) >> 15
        t2 = cand ^ ((cand ^ t2) & m)

    thr = (t1.astype(jnp.int32) << 16) | ((t2.astype(jnp.int32) ^ 0x8000) & 0xFFFF)

    gt = abits > thr
    eq = abits == thr
    cnt_gt = _sum_sublanes(gt.astype(jnp.int32))
    need = (TOPK - cnt_gt).astype(jnp.float32)
    rank = jax.lax.dot_general(
        tri_ref[...], eq.astype(jnp.float32), (((1,), (0,)), ((), ())),
        preferred_element_type=jnp.float32)
    mask = gt | (eq & (rank < need))

    gated_t = jnp.where(mask, latent_t, 0.0)

    @pl.when(pl.program_id(0) == 0)
    def _():
        alpha = alpha_ref[0, 0]
        w_ref[...] = prevw_ref[...] + alpha * (decw_ref[...] - prevw_ref[...])
    w = w_ref[...]

    mod_ref[...] = jax.lax.dot_general(
        gated_t, w, (((0,), (1,)), ((), ())), preferred_element_type=jnp.float32)


def kernel(x, prev_weight, enc_W, enc_b, dec_W, alpha):
    B, L, D = x.shape
    Dl = enc_W.shape[0]
    N = B * L
    R = ROW_BLOCK
    x_flat = x.reshape(N, D)
    ll = jnp.arange(Dl, dtype=jnp.int32)
    tri = (ll[None, :] < ll[:, None]).astype(jnp.float32)
    mod_flat, latent, W = pl.pallas_call(
        _fused_kernel,
        grid=(N // R,),
        in_specs=[
            pl.BlockSpec((R, D), lambda i: (i, 0)),
            pl.BlockSpec((Dl, D), lambda i: (0, 0)),
            pl.BlockSpec((Dl, 1), lambda i: (0, 0)),
            pl.BlockSpec((D, Dl), lambda i: (0, 0)),
            pl.BlockSpec((D, Dl), lambda i: (0, 0)),
            pl.BlockSpec((1, 1), lambda i: (0, 0)),
            pl.BlockSpec((Dl, Dl), lambda i: (0, 0)),
        ],
        out_specs=[
            pl.BlockSpec((R, D), lambda i: (i, 0)),
            pl.BlockSpec((R, Dl), lambda i: (i, 0)),
            pl.BlockSpec((D, Dl), lambda i: (0, 0)),
        ],
        out_shape=[
            jax.ShapeDtypeStruct((N, D), jnp.float32),
            jax.ShapeDtypeStruct((N, Dl), jnp.float32),
            jax.ShapeDtypeStruct((D, Dl), jnp.float32),
        ],
    )(x_flat, enc_W, enc_b.reshape(Dl, 1), prev_weight, dec_W,
      jnp.asarray(alpha, jnp.float32).reshape(1, 1), tri)
    return (mod_flat.reshape(B, L, D), latent, W)


# speedup vs baseline: 39.6714x; 1.0530x over previous
import jax
import jax.numpy as jnp
import numpy as np
from jax.experimental import pallas as pl

TOPK = 50
ROW_BLOCK = 2048


def _sum_sublanes(v):
    while v.shape[0] > 16:
        h = v.shape[0] // 2
        v = v[:h] + v[h:]
    return jnp.sum(v, axis=0, keepdims=True)


def _fused_kernel(x_ref, encw_ref, encb_ref, prevw_ref, decw_ref, alpha_ref,
                  tri_ref, mod_ref, latent_ref, w_ref):
    x = x_ref[...]
    encw = encw_ref[...]
    latent_t = jax.lax.dot_general(
        encw, x, (((1,), (1,)), ((), ())), preferred_element_type=jnp.float32)
    latent_t = latent_t + encb_ref[...]
    latent_ref[...] = latent_t.T

    abits = jax.lax.bitcast_convert_type(latent_t, jnp.int32) & jnp.int32(0x7FFFFFFF)
    rcols = abits.shape[1]

    hi = (abits >> 16).astype(jnp.int16)
    t1 = jnp.zeros((1, rcols), jnp.int16)
    for b in range(14, -1, -1):
        cand = t1 | jnp.int16(1 << b)
        cnt = _sum_sublanes((hi >= cand).astype(jnp.int16))
        m = (cnt - jnp.int16(TOPK)) >> 15
        t1 = cand ^ ((cand ^ t1) & m)
    cnt_hi_gt = _sum_sublanes((hi > t1).astype(jnp.int16))
    k2 = jnp.int16(TOPK) - cnt_hi_gt

    lo = (abits ^ jnp.int32(0x8000)).astype(jnp.int16)
    loa = jnp.where(hi == t1, lo, jnp.int16(-32768))
    t2 = jnp.full((1, rcols), -32768, dtype=jnp.int16)
    for b in range(15, -1, -1):
        cand = t2 + jnp.int16(np.int16(np.uint16(1 << b)))
        cnt = _sum_sublanes((loa >= cand).astype(jnp.int16))
        m = (cnt - k2) >> 15
        t2 = cand ^ ((cand ^ t2) & m)

    thr = (t1.astype(jnp.int32) << 16) | ((t2.astype(jnp.int32) ^ 0x8000) & 0xFFFF)

    gt = abits > thr
    eq = abits == thr
    cnt_gt = _sum_sublanes(gt.astype(jnp.int32))
    need = (TOPK - cnt_gt).astype(jnp.float32)
    rank = jax.lax.dot_general(
        tri_ref[...], eq.astype(jnp.float32), (((1,), (0,)), ((), ())),
        preferred_element_type=jnp.float32)
    mask = gt | (eq & (rank < need))

    gated_t = jnp.where(mask, latent_t, 0.0)

    @pl.when(pl.program_id(0) == 0)
    def _():
        alpha = alpha_ref[0, 0]
        w_ref[...] = prevw_ref[...] + alpha * (decw_ref[...] - prevw_ref[...])
    w = w_ref[...]

    mod_ref[...] = jax.lax.dot_general(
        gated_t, w, (((0,), (1,)), ((), ())), preferred_element_type=jnp.float32)


def kernel(x, prev_weight, enc_W, enc_b, dec_W, alpha):
    B, L, D = x.shape
    Dl = enc_W.shape[0]
    N = B * L
    R = ROW_BLOCK
    x_flat = x.reshape(N, D)
    ll = jnp.arange(Dl, dtype=jnp.int32)
    tri = (ll[None, :] < ll[:, None]).astype(jnp.float32)
    mod_flat, latent, W = pl.pallas_call(
        _fused_kernel,
        grid=(N // R,),
        in_specs=[
            pl.BlockSpec((R, D), lambda i: (i, 0)),
            pl.BlockSpec((Dl, D), lambda i: (0, 0)),
            pl.BlockSpec((Dl, 1), lambda i: (0, 0)),
            pl.BlockSpec((D, Dl), lambda i: (0, 0)),
            pl.BlockSpec((D, Dl), lambda i: (0, 0)),
            pl.BlockSpec((1, 1), lambda i: (0, 0)),
            pl.BlockSpec((Dl, Dl), lambda i: (0, 0)),
        ],
        out_specs=[
            pl.BlockSpec((R, D), lambda i: (i, 0)),
            pl.BlockSpec((R, Dl), lambda i: (i, 0)),
            pl.BlockSpec((D, Dl), lambda i: (0, 0)),
        ],
        out_shape=[
            jax.ShapeDtypeStruct((N, D), jnp.float32),
            jax.ShapeDtypeStruct((N, Dl), jnp.float32),
            jax.ShapeDtypeStruct((D, Dl), jnp.float32),
        ],
    )(x_flat, enc_W, enc_b.reshape(Dl, 1), prev_weight, dec_W,
      jnp.asarray(alpha, jnp.float32).reshape(1, 1), tri)
    return (mod_flat.reshape(B, L, D), latent, W)
